# baseline (speedup 1.0000x reference)
import jax
import jax.numpy as jnp
from jax.experimental import pallas as pl
from jax.experimental.pallas import tpu as pltpu

_K = 1024
_D = 64
_L = 8
_M = 512


def _rvq_body(x_ref, cb_ref, out_ref, idx_ref):
    x = x_ref[...]
    r = x
    out = jnp.zeros_like(x)
    for l in range(_L):
        cb = cb_ref[l]
        c2 = jnp.sum(cb * cb, axis=1)
        s = jax.lax.dot_general(
            r.astype(jnp.bfloat16), cb.astype(jnp.bfloat16),
            (((1,), (1,)), ((), ())),
            preferred_element_type=jnp.float32)
        r2 = jnp.sum(r * r, axis=1, keepdims=True)
        d2 = r2 - 2.0 * s + c2[None, :]
        idx = jnp.argmin(d2, axis=1)
        onehot = (jax.lax.broadcasted_iota(jnp.int32, (_M, _K), 1)
                  == idx[:, None]).astype(jnp.float32)
        q = jax.lax.dot_general(
            onehot, cb, (((1,), (0,)), ((), ())),
            preferred_element_type=jnp.float32,
            precision=jax.lax.Precision.HIGHEST)
        out = out + q
        r = r - q
        idx_ref[l, :] = idx
    out_ref[...] = out


def kernel(input, codebooks):
    B, D, T = input.shape
    L = codebooks.shape[0]
    N = B * T
    x = input.transpose(0, 2, 1).reshape(N, D)

    out_flat, idx_flat = pl.pallas_call(
        _rvq_body,
        grid=(N // _M,),
        in_specs=[
            pl.BlockSpec((_M, D), lambda i: (i, 0)),
            pl.BlockSpec((L, _K, D), lambda i: (0, 0, 0)),
        ],
        out_specs=[
            pl.BlockSpec((_M, D), lambda i: (i, 0)),
            pl.BlockSpec((L, _M), lambda i: (0, i)),
        ],
        out_shape=[
            jax.ShapeDtypeStruct((N, D), jnp.float32),
            jax.ShapeDtypeStruct((L, N), jnp.int32),
        ],
    )(x, codebooks)

    out = out_flat.reshape(B, T, D).transpose(0, 2, 1)
    indices = idx_flat.reshape(L, B, T).transpose(1, 0, 2).astype(jnp.int64)
    return out, indices


# speedup vs baseline: 2.6692x; 2.6692x over previous
import jax
import jax.numpy as jnp
from jax.experimental import pallas as pl
from jax.experimental.pallas import tpu as pltpu

_K = 1024
_D = 64
_L = 8
_M = 512
_G = 4
_KG = _K // _G
_GD = _G * _D


def _rvq_body(x_ref, cb_ref, cbg_ref, out_ref, idx_ref, c2_ref, cbs_ref):
    i = pl.program_id(0)

    @pl.when(i == 0)
    def _prep():
        for l in range(_L):
            cb = cb_ref[l]
            c2_ref[l, :] = jnp.sum(cb * cb, axis=1)
            rem = cbg_ref[l]
            for p in range(4):
                plane = rem.astype(jnp.bfloat16)
                cbs_ref[l, p, :, :] = plane
                rem = rem - plane.astype(jnp.float32)

    x = x_ref[...]
    r = x
    out = jnp.zeros_like(x)
    lane_g = jax.lax.broadcasted_iota(jnp.int32, (_M, _G * _KG), 1) % _KG
    lane_s = jax.lax.broadcasted_iota(jnp.int32, (_M, _GD), 1) // _D
    for l in range(_L):
        cb = cb_ref[l]
        s = jax.lax.dot_general(
            r.astype(jnp.bfloat16), cb.astype(jnp.bfloat16),
            (((1,), (1,)), ((), ())),
            preferred_element_type=jnp.float32)
        r2 = jnp.sum(r * r, axis=1, keepdims=True)
        d2 = r2 - 2.0 * s + c2_ref[l, :][None, :]
        idx = jnp.argmin(d2, axis=1)

        oh = (lane_g == (idx // _G)[:, None]).astype(jnp.bfloat16)
        planes = cbs_ref[l].reshape(4 * _KG, _GD)
        s1 = jax.lax.dot_general(
            oh, planes, (((1,), (0,)), ((), ())),
            preferred_element_type=jnp.float32)
        sel = (lane_s == (idx % _G)[:, None]).astype(jnp.float32)
        picked = s1 * sel
        q = (picked[:, 0 * _D:1 * _D] + picked[:, 1 * _D:2 * _D]
             + picked[:, 2 * _D:3 * _D] + picked[:, 3 * _D:4 * _D])
        out = out + q
        r = r - q
        idx_ref[l, :] = idx
    out_ref[...] = out


def kernel(input, codebooks):
    B, D, T = input.shape
    L = codebooks.shape[0]
    N = B * T
    x = input.transpose(0, 2, 1).reshape(N, D)
    cbg = codebooks.reshape(L, _KG, _GD)

    out_flat, idx_flat = pl.pallas_call(
        _rvq_body,
        grid=(N // _M,),
        in_specs=[
            pl.BlockSpec((_M, D), lambda i: (i, 0)),
            pl.BlockSpec((L, _K, D), lambda i: (0, 0, 0)),
            pl.BlockSpec((L, _KG, _GD), lambda i: (0, 0, 0)),
        ],
        out_specs=[
            pl.BlockSpec((_M, D), lambda i: (i, 0)),
            pl.BlockSpec((L, _M), lambda i: (0, i)),
        ],
        out_shape=[
            jax.ShapeDtypeStruct((N, D), jnp.float32),
            jax.ShapeDtypeStruct((L, N), jnp.int32),
        ],
        scratch_shapes=[
            pltpu.VMEM((_L, _K), jnp.float32),
            pltpu.VMEM((_L, 4, _KG, _GD), jnp.bfloat16),
        ],
    )(x, codebooks, cbg)

    out = out_flat.reshape(B, T, D).transpose(0, 2, 1)
    indices = idx_flat.reshape(L, B, T).transpose(1, 0, 2).astype(jnp.int64)
    return out, indices


# speedup vs baseline: 2.9028x; 1.0875x over previous
import jax
import jax.numpy as jnp
from jax.experimental import pallas as pl
from jax.experimental.pallas import tpu as pltpu

_K = 1024
_D = 64
_L = 8
_M = 512
_G = 4
_KG = _K // _G
_GD = _G * _D


def _rvq_body(x_ref, cb_ref, cbg_ref, out_ref, idx_ref, c2_ref, cbs_ref):
    i = pl.program_id(0)

    @pl.when(i == 0)
    def _prep():
        for l in range(_L):
            cb = cb_ref[l]
            c2_ref[l, :] = jnp.sum(cb * cb, axis=1)
            rem = cbg_ref[l]
            for p in range(3):
                plane = rem.astype(jnp.bfloat16)
                cbs_ref[l, p, :, :] = plane
                rem = rem - plane.astype(jnp.float32)

    x = x_ref[...]
    r = x
    out = jnp.zeros_like(x)
    lane_g = jax.lax.broadcasted_iota(jnp.int32, (_M, 3 * _KG), 1) % _KG
    lane_s = jax.lax.broadcasted_iota(jnp.int32, (_M, _GD), 1) // _D
    for l in range(_L):
        cb = cb_ref[l]
        s = jax.lax.dot_general(
            r.astype(jnp.bfloat16), cb.astype(jnp.bfloat16),
            (((1,), (1,)), ((), ())),
            preferred_element_type=jnp.float32)
        r2 = jnp.sum(r * r, axis=1, keepdims=True)
        d2 = r2 - 2.0 * s + c2_ref[l, :][None, :]
        idx = jnp.argmin(d2, axis=1)

        oh = (lane_g == (idx // _G)[:, None]).astype(jnp.bfloat16)
        planes = cbs_ref[l].reshape(3 * _KG, _GD)
        s1 = jax.lax.dot_general(
            oh, planes, (((1,), (0,)), ((), ())),
            preferred_element_type=jnp.float32)
        sel = (lane_s == (idx % _G)[:, None]).astype(jnp.float32)
        picked = s1 * sel
        q = (picked[:, 0 * _D:1 * _D] + picked[:, 1 * _D:2 * _D]
             + picked[:, 2 * _D:3 * _D] + picked[:, 3 * _D:4 * _D])
        out = out + q
        r = r - q
        idx_ref[l, :] = idx
    out_ref[...] = out


def kernel(input, codebooks):
    B, D, T = input.shape
    L = codebooks.shape[0]
    N = B * T
    x = input.transpose(0, 2, 1).reshape(N, D)
    cbg = codebooks.reshape(L, _KG, _GD)

    out_flat, idx_flat = pl.pallas_call(
        _rvq_body,
        grid=(N // _M,),
        in_specs=[
            pl.BlockSpec((_M, D), lambda i: (i, 0)),
            pl.BlockSpec((L, _K, D), lambda i: (0, 0, 0)),
            pl.BlockSpec((L, _KG, _GD), lambda i: (0, 0, 0)),
        ],
        out_specs=[
            pl.BlockSpec((_M, D), lambda i: (i, 0)),
            pl.BlockSpec((L, _M), lambda i: (0, i)),
        ],
        out_shape=[
            jax.ShapeDtypeStruct((N, D), jnp.float32),
            jax.ShapeDtypeStruct((L, N), jnp.int32),
        ],
        scratch_shapes=[
            pltpu.VMEM((_L, _K), jnp.float32),
            pltpu.VMEM((_L, 3, _KG, _GD), jnp.bfloat16),
        ],
    )(x, codebooks, cbg)

    out = out_flat.reshape(B, T, D).transpose(0, 2, 1)
    indices = idx_flat.reshape(L, B, T).transpose(1, 0, 2).astype(jnp.int64)
    return out, indices


# speedup vs baseline: 2.9903x; 1.0301x over previous
import jax
import jax.numpy as jnp
from jax.experimental import pallas as pl
from jax.experimental.pallas import tpu as pltpu

_K = 1024
_D = 64
_L = 8
_M = 512
_G = 4
_KG = _K // _G
_GD = _G * _D


def _rvq_body(x_ref, cb_ref, cbg_ref, out_ref, idx_ref, c2_ref, cbs_ref,
              cbb_ref):
    b = pl.program_id(0)
    t = pl.program_id(1)

    @pl.when((b == 0) & (t == 0))
    def _prep():
        for l in range(_L):
            cb = cb_ref[l]
            c2_ref[l, :] = jnp.sum(cb * cb, axis=1)
            cbb_ref[l, :, :] = cb.astype(jnp.bfloat16)
            rem = cbg_ref[l]
            for p in range(3):
                plane = rem.astype(jnp.bfloat16)
                cbs_ref[l, p, :, :] = plane
                rem = rem - plane.astype(jnp.float32)

    x = x_ref[0].T
    r = x
    out = jnp.zeros_like(x)
    lane_g = jax.lax.broadcasted_iota(jnp.int32, (_M, 3 * _KG), 1) % _KG
    lane_s = jax.lax.broadcasted_iota(jnp.int32, (_M, _GD), 1) // _D
    for l in range(_L):
        s = jax.lax.dot_general(
            r.astype(jnp.bfloat16), cbb_ref[l],
            (((1,), (1,)), ((), ())),
            preferred_element_type=jnp.float32)
        r2 = jnp.sum(r * r, axis=1, keepdims=True)
        d2 = r2 - 2.0 * s + c2_ref[l, :][None, :]
        idx = jnp.argmin(d2, axis=1)

        oh = (lane_g == (idx // _G)[:, None]).astype(jnp.bfloat16)
        planes = cbs_ref[l].reshape(3 * _KG, _GD)
        s1 = jax.lax.dot_general(
            oh, planes, (((1,), (0,)), ((), ())),
            preferred_element_type=jnp.float32)
        sel = (lane_s == (idx % _G)[:, None]).astype(jnp.float32)
        picked = s1 * sel
        q = (picked[:, 0 * _D:1 * _D] + picked[:, 1 * _D:2 * _D]
             + picked[:, 2 * _D:3 * _D] + picked[:, 3 * _D:4 * _D])
        out = out + q
        r = r - q
        idx_ref[0, l, :] = idx
    out_ref[0] = out.T


def kernel(input, codebooks):
    B, D, T = input.shape
    L = codebooks.shape[0]
    cbg = codebooks.reshape(L, _KG, _GD)

    out, idx = pl.pallas_call(
        _rvq_body,
        grid=(B, T // _M),
        in_specs=[
            pl.BlockSpec((1, D, _M), lambda b, t: (b, 0, t)),
            pl.BlockSpec((L, _K, D), lambda b, t: (0, 0, 0)),
            pl.BlockSpec((L, _KG, _GD), lambda b, t: (0, 0, 0)),
        ],
        out_specs=[
            pl.BlockSpec((1, D, _M), lambda b, t: (b, 0, t)),
            pl.BlockSpec((1, L, _M), lambda b, t: (b, 0, t)),
        ],
        out_shape=[
            jax.ShapeDtypeStruct((B, D, T), jnp.float32),
            jax.ShapeDtypeStruct((B, L, T), jnp.int32),
        ],
        scratch_shapes=[
            pltpu.VMEM((_L, _K), jnp.float32),
            pltpu.VMEM((_L, 3, _KG, _GD), jnp.bfloat16),
            pltpu.VMEM((_L, _K, _D), jnp.bfloat16),
        ],
    )(input, codebooks, cbg)

    return out, idx.astype(jnp.int64)
